# Initial kernel scaffold; baseline (speedup 1.0000x reference)
import jax
import jax.numpy as jnp
from jax.experimental import pallas as pl


def kernel(x, edge_index, subset_indices, Ws, bs):
    raise NotImplementedError("write your pallas kernel here")


# speedup vs baseline: 6.7429x; 6.7429x over previous
import functools

import jax
import jax.numpy as jnp
from jax import lax
from jax.experimental import pallas as pl
from jax.experimental.pallas import tpu as pltpu, tpu_sc as plsc

N = 10000
D = 128
N_PAD = 10240
NC, NS = 2, 16
NW = NC * NS
CH = 128
ROWS_PER_SUB = N_PAD // NS
DW = 16
BR = 512
GRID = N_PAD // BR

_mesh = plsc.VectorSubcoreMesh(core_axis_name="c", subcore_axis_name="s")


def _deg_body(dst_hbm, zeros_hbm, ones_hbm, out_hbm, dst_v, ones_v, deg_sh, sem):
    c = lax.axis_index("c")
    s = lax.axis_index("s")
    n_chunks = dst_hbm.shape[0] // NW // CH
    base = (s * NC + c) * (n_chunks * CH)
    pltpu.sync_copy(zeros_hbm.at[pl.ds(s * ROWS_PER_SUB, ROWS_PER_SUB)],
                    deg_sh.at[pl.ds(s * ROWS_PER_SUB, ROWS_PER_SUB)])
    pltpu.sync_copy(ones_hbm, ones_v)
    plsc.subcore_barrier()

    def body(j, carry):
        off = base + j * CH
        pltpu.sync_copy(dst_hbm.at[pl.ds(off, CH)], dst_v)
        pltpu.sync_copy(ones_v, deg_sh.at[dst_v], add=True)
        return carry

    lax.fori_loop(0, n_chunks, body, 0)
    plsc.subcore_barrier()
    pltpu.sync_copy(deg_sh.at[pl.ds(s * ROWS_PER_SUB, ROWS_PER_SUB)],
                    out_hbm.at[c, pl.ds(s * ROWS_PER_SUB, ROWS_PER_SUB)])


def _sc_degree(dst_pad, zeros_nd, ones_cd):
    k = pl.kernel(
        _deg_body,
        out_type=jax.ShapeDtypeStruct((NC, N_PAD, D), jnp.float32),
        mesh=_mesh,
        scratch_types=[
            pltpu.VMEM((CH,), jnp.int32),
            pltpu.VMEM((CH, D), jnp.float32),
            pltpu.VMEM_SHARED((N_PAD, D), jnp.float32),
            pltpu.SemaphoreType.DMA,
        ],
    )
    return k(dst_pad, zeros_nd, ones_cd)


def _scatter_body(g_hbm, src_hbm, dst_hbm, zeros_hbm, out_hbm,
                  src_v, dst_v, rows_v, acc_sh, sem):
    c = lax.axis_index("c")
    s = lax.axis_index("s")
    n_chunks = src_hbm.shape[0] // NW // CH
    base = (s * NC + c) * (n_chunks * CH)
    rsl = pl.ds(s * ROWS_PER_SUB, ROWS_PER_SUB)

    @pl.when(c == 0)
    def _():
        pltpu.sync_copy(g_hbm.at[rsl], acc_sh.at[rsl])

    @pl.when(c != 0)
    def _():
        pltpu.sync_copy(zeros_hbm.at[rsl], acc_sh.at[rsl])

    plsc.subcore_barrier()

    def body(j, carry):
        off = base + j * CH
        pltpu.sync_copy(src_hbm.at[pl.ds(off, CH)], src_v)
        pltpu.sync_copy(dst_hbm.at[pl.ds(off, CH)], dst_v)
        pltpu.async_copy(g_hbm.at[src_v], rows_v, sem).wait()
        pltpu.sync_copy(rows_v, acc_sh.at[dst_v], add=True)
        return carry

    lax.fori_loop(0, n_chunks, body, 0)
    plsc.subcore_barrier()
    pltpu.sync_copy(acc_sh.at[rsl], out_hbm.at[c, rsl])


def _sc_scatter(g, src_pad, dst_pad, zeros_nd):
    k = pl.kernel(
        _scatter_body,
        out_type=jax.ShapeDtypeStruct((NC, N_PAD, D), jnp.float32),
        mesh=_mesh,
        scratch_types=[
            pltpu.VMEM((CH,), jnp.int32),
            pltpu.VMEM((CH,), jnp.int32),
            pltpu.VMEM((CH, D), jnp.float32),
            pltpu.VMEM_SHARED((N_PAD, D), jnp.float32),
            pltpu.SemaphoreType.DMA,
        ],
    )
    return k(g, src_pad, dst_pad, zeros_nd)


def _gather_body(s_hbm, idx_hbm, out_hbm, idx_v, vals_v, sem):
    c = lax.axis_index("c")
    s = lax.axis_index("s")
    b = idx_hbm.shape[0] // NW
    base = (s * NC + c) * b
    pltpu.sync_copy(idx_hbm.at[pl.ds(base, b)], idx_v)
    pltpu.async_copy(s_hbm.at[idx_v], vals_v, sem).wait()
    pltpu.sync_copy(vals_v, out_hbm.at[pl.ds(base, b)])


def _sc_subset_gather(s_flat, subset_indices):
    sub = subset_indices.shape[0]
    b = sub // NW
    k = pl.kernel(
        _gather_body,
        out_type=jax.ShapeDtypeStruct((sub,), jnp.float32),
        mesh=_mesh,
        scratch_types=[
            pltpu.VMEM((b,), jnp.int32),
            pltpu.VMEM((b,), jnp.float32),
            pltpu.SemaphoreType.DMA,
        ],
    )
    return k(s_flat, subset_indices)


def _dis_col(deg_ref):
    d = deg_ref[0][:, 0:1] + deg_ref[1][:, 0:1] + 1.0
    return lax.rsqrt(d)


def _first_layer_body(x_ref, deg_ref, w_ref, o_ref):
    dis = _dis_col(deg_ref)
    hw = jnp.dot(x_ref[...], w_ref[...], preferred_element_type=jnp.float32,
                 precision=lax.Precision.HIGHEST)
    o_ref[...] = hw * dis


def _tc_first_layer(x_pad, deg, w0):
    return pl.pallas_call(
        _first_layer_body,
        grid=(GRID,),
        in_specs=[
            pl.BlockSpec((BR, D), lambda i: (i, 0)),
            pl.BlockSpec((NC, BR, D), lambda i: (0, i, 0)),
            pl.BlockSpec((D, D), lambda i: (0, 0)),
        ],
        out_specs=pl.BlockSpec((BR, D), lambda i: (i, 0)),
        out_shape=jax.ShapeDtypeStruct((N_PAD, D), jnp.float32),
    )(x_pad, deg, w0)


def _mid_layer_body(acc_ref, deg_ref, w_ref, b_ref, o_ref):
    dis = _dis_col(deg_ref)
    h = jnp.maximum((acc_ref[0] + acc_ref[1]) * dis + b_ref[...], 0.0)
    hw = jnp.dot(h, w_ref[...], preferred_element_type=jnp.float32,
                 precision=lax.Precision.HIGHEST)
    o_ref[...] = hw * dis


def _tc_mid_layer(acc, deg, w, b_row):
    return pl.pallas_call(
        _mid_layer_body,
        grid=(GRID,),
        in_specs=[
            pl.BlockSpec((NC, BR, D), lambda i: (0, i, 0)),
            pl.BlockSpec((NC, BR, D), lambda i: (0, i, 0)),
            pl.BlockSpec((D, D), lambda i: (0, 0)),
            pl.BlockSpec((1, D), lambda i: (0, 0)),
        ],
        out_specs=pl.BlockSpec((BR, D), lambda i: (i, 0)),
        out_shape=jax.ShapeDtypeStruct((N_PAD, D), jnp.float32),
    )(acc, deg, w, b_row)


def _rowsum_body(acc_ref, deg_ref, b_ref, o_ref):
    dis = _dis_col(deg_ref)
    h = (acc_ref[0] + acc_ref[1]) * dis + b_ref[...]
    o_ref[...] = jnp.sum(h, axis=1, keepdims=True)


def _tc_rowsum(acc, deg, b_row):
    return pl.pallas_call(
        _rowsum_body,
        grid=(GRID,),
        in_specs=[
            pl.BlockSpec((NC, BR, D), lambda i: (0, i, 0)),
            pl.BlockSpec((NC, BR, D), lambda i: (0, i, 0)),
            pl.BlockSpec((1, D), lambda i: (0, 0)),
        ],
        out_specs=pl.BlockSpec((BR, 1), lambda i: (i, 0)),
        out_shape=jax.ShapeDtypeStruct((N_PAD, 1), jnp.float32),
    )(acc, deg, b_row)


def _argmax_body(s_ref, max_ref, arg_ref):
    v = s_ref[...]
    m = jnp.max(v)
    r, cdim = v.shape
    flat = (lax.broadcasted_iota(jnp.int32, (r, cdim), 0) * cdim
            + lax.broadcasted_iota(jnp.int32, (r, cdim), 1))
    arg = jnp.min(jnp.where(v == m, flat, jnp.int32(2**30)))
    max_ref[0, 0] = m
    arg_ref[0, 0] = arg


def _tc_max_argmax(s_sub2d):
    return pl.pallas_call(
        _argmax_body,
        out_specs=(pl.BlockSpec(memory_space=pltpu.SMEM),
                   pl.BlockSpec(memory_space=pltpu.SMEM)),
        out_shape=(jax.ShapeDtypeStruct((1, 1), jnp.float32),
                   jax.ShapeDtypeStruct((1, 1), jnp.int32)),
    )(s_sub2d)


def kernel(x, edge_index, subset_indices, Ws, bs):
    num_layers = Ws.shape[0]
    e = edge_index.shape[1]
    e_pad = ((e + NW * CH - 1) // (NW * CH)) * (NW * CH)

    x_pad = jnp.zeros((N_PAD, D), jnp.float32).at[:N].set(x)
    src_pad = jnp.zeros((e_pad,), jnp.int32).at[:e].set(edge_index[0])
    dst_pad = jnp.full((e_pad,), N, jnp.int32).at[:e].set(edge_index[1])
    zeros_nd = jnp.zeros((N_PAD, D), jnp.float32)
    ones_cd = jnp.ones((CH, D), jnp.float32)

    deg = _sc_degree(dst_pad, zeros_nd, ones_cd)

    g = _tc_first_layer(x_pad, deg, Ws[0])
    for i in range(num_layers - 1):
        acc = _sc_scatter(g, src_pad, dst_pad, zeros_nd)
        g = _tc_mid_layer(acc, deg, Ws[i + 1], bs[i].reshape(1, D))
    acc = _sc_scatter(g, src_pad, dst_pad, zeros_nd)
    s_all = _tc_rowsum(acc, deg, bs[num_layers - 1].reshape(1, D))

    s_sub = _sc_subset_gather(s_all.reshape(N_PAD), subset_indices)
    smax, sarg = _tc_max_argmax(s_sub.reshape(2, -1))
    return (smax[0, 0], sarg[0, 0])
